# baseline (speedup 1.0000x reference)
import functools

import jax
import jax.numpy as jnp
from jax import lax
from jax.experimental import pallas as pl
from jax.experimental.pallas import tpu as pltpu
from jax.experimental.pallas import tpu_sc as plsc

N_NODES = 10000
IN_CH = 128
HID_CH = 128
OUT_CH = 64
P_HOPS = 10

NPAD = 10240
TRASH = 10100
N_EDGES = 320000
NTILES = 32
CHUNK = 128
CPT = 81
NBUF = 3
GRP = 8
EPT = CHUNK * CPT
E_PAD = NTILES * EPT
NB = 512
GRID = NPAD // NB
RPT = NPAD // 16
assert CPT % NBUF == 0

def _sc_deg_body(cidx_hbm, ones_hbm, zeros_hbm, out_hbm, cidx_v, ones_v, deg_sh):
    c = lax.axis_index("c")
    s = lax.axis_index("s")
    t = c * 16 + s
    pltpu.sync_copy(zeros_hbm.at[pl.ds(s * RPT, RPT)], deg_sh.at[pl.ds(s * RPT, RPT)])
    pltpu.sync_copy(ones_hbm, ones_v)
    pltpu.sync_copy(cidx_hbm.at[t], cidx_v)
    plsc.subcore_barrier()

    def chunk(j, carry):
        pltpu.sync_copy(ones_v, deg_sh.at[cidx_v.at[j]], add=True)
        return carry

    lax.fori_loop(0, CPT, chunk, 0)
    plsc.subcore_barrier()
    pltpu.sync_copy(deg_sh.at[pl.ds(s * RPT, RPT)], out_hbm.at[c, pl.ds(s * RPT, RPT)])


@functools.cache
def _get_sc_deg():
    return pl.kernel(
        _sc_deg_body,
        out_type=jax.ShapeDtypeStruct((2, NPAD, 8), jnp.float32),
        mesh=plsc.VectorSubcoreMesh(core_axis_name="c", subcore_axis_name="s"),
        compiler_params=pltpu.CompilerParams(use_tc_tiling_on_sc=False),
        scratch_types=[
            pltpu.VMEM((CPT, CHUNK), jnp.int32),
            pltpu.VMEM((CHUNK, 8), jnp.float32),
            pltpu.VMEM_SHARED((NPAD, 8), jnp.float32),
        ],
    )


def _sc_hop_body(g_hbm, ridx_hbm, cidx_hbm, zeros_hbm, out_hbm,
                 ridx_v, cidx_v, rows_v, g_sh, accum_sh, sem):
    c = lax.axis_index("c")
    s = lax.axis_index("s")
    t = c * 16 + s
    pltpu.sync_copy(g_hbm.at[pl.ds(s * RPT, RPT)], g_sh.at[pl.ds(s * RPT, RPT)])
    pltpu.sync_copy(zeros_hbm.at[pl.ds(s * RPT, RPT)], accum_sh.at[pl.ds(s * RPT, RPT)])
    pltpu.sync_copy(ridx_hbm.at[t], ridx_v)
    pltpu.sync_copy(cidx_hbm.at[t], cidx_v)
    plsc.subcore_barrier()

    def gsrc(b):
        return g_hbm if b == NBUF - 1 else g_sh

    for b in range(NBUF):
        pltpu.async_copy(gsrc(b).at[ridx_v.at[b]], rows_v.at[b], sem.at[b])

    def chunk(j0, carry):
        for b in range(NBUF):
            j = j0 * NBUF + b
            pltpu.make_async_copy(gsrc(b).at[ridx_v.at[j]], rows_v.at[b],
                                  sem.at[b]).wait()
            pltpu.sync_copy(rows_v.at[b], accum_sh.at[cidx_v.at[j]], add=True)
            pltpu.async_copy(gsrc(b).at[ridx_v.at[j + NBUF]], rows_v.at[b],
                             sem.at[b])
        return carry

    lax.fori_loop(0, CPT // NBUF - 1, chunk, 0)
    for b in range(NBUF):
        j = CPT - NBUF + b
        pltpu.make_async_copy(gsrc(b).at[ridx_v.at[j]], rows_v.at[b],
                              sem.at[b]).wait()
        pltpu.sync_copy(rows_v.at[b], accum_sh.at[cidx_v.at[j]], add=True)
    plsc.subcore_barrier()
    pltpu.sync_copy(accum_sh.at[pl.ds(s * RPT, RPT)], out_hbm.at[c, pl.ds(s * RPT, RPT)])


@functools.cache
def _get_sc_hop():
    return pl.kernel(
        _sc_hop_body,
        out_type=jax.ShapeDtypeStruct((2, NPAD, OUT_CH), jnp.float32),
        mesh=plsc.VectorSubcoreMesh(core_axis_name="c", subcore_axis_name="s"),
        compiler_params=pltpu.CompilerParams(use_tc_tiling_on_sc=False),
        scratch_types=[
            pltpu.VMEM((CPT, CHUNK), jnp.int32),
            pltpu.VMEM((CPT, CHUNK), jnp.int32),
            pltpu.VMEM((NBUF, CHUNK, OUT_CH), jnp.float32),
            pltpu.VMEM_SHARED((NPAD, OUT_CH), jnp.float32),
            pltpu.VMEM_SHARED((NPAD, OUT_CH), jnp.float32),
            pltpu.SemaphoreType.DMA((NBUF,)),
        ],
    )


def _mlp_body(x_ref, w1t_ref, b1_ref, w2t_ref, b2_ref, degp_ref,
              g0_ref, dis2_ref, disinv_ref):
    h1 = jnp.dot(x_ref[...], w1t_ref[...], preferred_element_type=jnp.float32)
    h1 = jnp.maximum(h1 + b1_ref[...], 0.0)
    h0 = jnp.dot(h1, w2t_ref[...], preferred_element_type=jnp.float32) + b2_ref[...]
    deg = 1.0 + degp_ref[0, :, 0:1] + degp_ref[1, :, 0:1]
    dis = lax.rsqrt(deg)
    g0_ref[...] = dis * h0
    dis2_ref[...] = jnp.broadcast_to(1.0 / deg, (NB, OUT_CH))
    disinv_ref[...] = jnp.broadcast_to(jnp.sqrt(deg), (NB, OUT_CH))


def _mlp_call(x_pad, w1t, b1, w2t, b2, degp):
    return pl.pallas_call(
        _mlp_body,
        grid=(GRID,),
        in_specs=[
            pl.BlockSpec((NB, IN_CH), lambda i: (i, 0)),
            pl.BlockSpec((IN_CH, HID_CH), lambda i: (0, 0)),
            pl.BlockSpec((1, HID_CH), lambda i: (0, 0)),
            pl.BlockSpec((HID_CH, OUT_CH), lambda i: (0, 0)),
            pl.BlockSpec((1, OUT_CH), lambda i: (0, 0)),
            pl.BlockSpec((2, NB, 8), lambda i: (0, i, 0)),
        ],
        out_specs=[
            pl.BlockSpec((NB, OUT_CH), lambda i: (i, 0)),
            pl.BlockSpec((NB, OUT_CH), lambda i: (i, 0)),
            pl.BlockSpec((NB, OUT_CH), lambda i: (i, 0)),
        ],
        out_shape=[
            jax.ShapeDtypeStruct((NPAD, OUT_CH), jnp.float32),
            jax.ShapeDtypeStruct((NPAD, OUT_CH), jnp.float32),
            jax.ShapeDtypeStruct((NPAD, OUT_CH), jnp.float32),
        ],
    )(x_pad, w1t, b1, w2t, b2, degp)


def _combine_body(acc_ref, g_ref, dis2_ref, g_out_ref):
    g_out_ref[...] = dis2_ref[...] * (acc_ref[0] + acc_ref[1] + g_ref[...])


def _combine_call(acc, g, dis2):
    return pl.pallas_call(
        _combine_body,
        grid=(GRID,),
        in_specs=[
            pl.BlockSpec((2, NB, OUT_CH), lambda i: (0, i, 0)),
            pl.BlockSpec((NB, OUT_CH), lambda i: (i, 0)),
            pl.BlockSpec((NB, OUT_CH), lambda i: (i, 0)),
        ],
        out_specs=pl.BlockSpec((NB, OUT_CH), lambda i: (i, 0)),
        out_shape=jax.ShapeDtypeStruct((NPAD, OUT_CH), jnp.float32),
    )(acc, g, dis2)


def _final_body(coes_ref, acc_ref, dis2_ref, disinv_ref, *refs):
    g_refs = refs[:P_HOPS]
    o_ref = refs[P_HOPS]
    g_last = g_refs[P_HOPS - 1][...]
    g_end = dis2_ref[...] * (acc_ref[0] + acc_ref[1] + g_last)
    acc_h = coes_ref[P_HOPS] * g_end
    for q in range(P_HOPS):
        acc_h = acc_h + coes_ref[q] * g_refs[q][...]
    h = disinv_ref[...] * acc_h
    m = jnp.max(h, axis=1, keepdims=True)
    e = jnp.exp(h - m)
    o_ref[...] = h - m - jnp.log(jnp.sum(e, axis=1, keepdims=True))


def _final_call(coes, acc, dis2, disinv, g_list):
    blk = pl.BlockSpec((NB, OUT_CH), lambda i: (i, 0))
    return pl.pallas_call(
        _final_body,
        grid=(GRID,),
        in_specs=[
            pl.BlockSpec(memory_space=pltpu.SMEM),
            pl.BlockSpec((2, NB, OUT_CH), lambda i: (0, i, 0)),
            blk, blk,
        ] + [blk] * P_HOPS,
        out_specs=blk,
        out_shape=jax.ShapeDtypeStruct((NPAD, OUT_CH), jnp.float32),
    )(coes, acc, dis2, disinv, *g_list)


def kernel(x, edge_index, W1, b1, W2, b2, coes):
    row = edge_index[0].astype(jnp.int32)
    col = edge_index[1].astype(jnp.int32)
    pad = E_PAD - N_EDGES
    row_p = jnp.concatenate([row, jnp.zeros((pad,), jnp.int32)]).reshape(
        NTILES, CPT, CHUNK)
    col_p = jnp.concatenate([col, jnp.full((pad,), TRASH, jnp.int32)]).reshape(
        NTILES, CPT, CHUNK)
    zeros8 = jnp.zeros((NPAD, 8), jnp.float32)
    ones8 = jnp.ones((CHUNK, 8), jnp.float32)
    zeros64 = jnp.zeros((NPAD, OUT_CH), jnp.float32)
    x_pad = jnp.zeros((NPAD, IN_CH), jnp.float32).at[:N_NODES].set(x)

    degp = _get_sc_deg()(col_p, ones8, zeros8)
    g, dis2, disinv = _mlp_call(x_pad, W1.T, b1.reshape(1, -1),
                                W2.T, b2.reshape(1, -1), degp)
    g_list = [g]
    for p in range(P_HOPS):
        acc = _get_sc_hop()(g, row_p, col_p, zeros64)
        if p < P_HOPS - 1:
            g = _combine_call(acc, g, dis2)
            g_list.append(g)
    out = _final_call(coes, acc, dis2, disinv, g_list)
    return out[:N_NODES]


# speedup vs baseline: 1.5242x; 1.5242x over previous
import functools

import jax
import jax.numpy as jnp
from jax import lax
from jax.experimental import pallas as pl
from jax.experimental.pallas import tpu as pltpu
from jax.experimental.pallas import tpu_sc as plsc

N_NODES = 10000
IN_CH = 128
HID_CH = 128
OUT_CH = 64
P_HOPS = 10

NPAD = 10240
TRASH = 10100
N_EDGES = 320000
NTILES = 32
CHUNK = 128
CPT = 81
NBUF = 3
GRP = 8
EPT = CHUNK * CPT
E_PAD = NTILES * EPT
NB = 512
GRID = NPAD // NB
RPT = NPAD // 16
assert CPT % NBUF == 0

def _sc_deg_body(cidx_hbm, ones_hbm, zeros_hbm, out_hbm, cidx_v, ones_v, deg_sh):
    c = lax.axis_index("c")
    s = lax.axis_index("s")
    t = c * 16 + s
    pltpu.sync_copy(zeros_hbm.at[pl.ds(s * RPT, RPT)], deg_sh.at[pl.ds(s * RPT, RPT)])
    pltpu.sync_copy(ones_hbm, ones_v)
    pltpu.sync_copy(cidx_hbm.at[t], cidx_v)
    plsc.subcore_barrier()

    def chunk(j, carry):
        pltpu.sync_copy(ones_v, deg_sh.at[cidx_v.at[j]], add=True)
        return carry

    lax.fori_loop(0, CPT, chunk, 0)
    plsc.subcore_barrier()
    pltpu.sync_copy(deg_sh.at[pl.ds(s * RPT, RPT)], out_hbm.at[c, pl.ds(s * RPT, RPT)])


@functools.cache
def _get_sc_deg():
    return pl.kernel(
        _sc_deg_body,
        out_type=jax.ShapeDtypeStruct((2, NPAD, 8), jnp.float32),
        mesh=plsc.VectorSubcoreMesh(core_axis_name="c", subcore_axis_name="s"),
        compiler_params=pltpu.CompilerParams(use_tc_tiling_on_sc=False),
        scratch_types=[
            pltpu.VMEM((CPT, CHUNK), jnp.int32),
            pltpu.VMEM((CHUNK, 8), jnp.float32),
            pltpu.VMEM_SHARED((NPAD, 8), jnp.float32),
        ],
    )


def _sc_hop_body(g_hbm, ridx_hbm, cidx_hbm, zeros_hbm, out_hbm,
                 ridx_v, cidx_v, rows_v, g_sh, accum_sh, sem, psem):
    c = lax.axis_index("c")
    s = lax.axis_index("s")
    t = c * 16 + s
    rs = pl.ds(s * RPT, RPT)
    pltpu.async_copy(g_hbm.at[rs], g_sh.at[rs], psem.at[0])
    pltpu.async_copy(zeros_hbm.at[rs], accum_sh.at[rs], psem.at[1])
    pltpu.async_copy(ridx_hbm.at[t], ridx_v, psem.at[2])
    pltpu.async_copy(cidx_hbm.at[t], cidx_v, psem.at[3])
    pltpu.make_async_copy(g_hbm.at[rs], g_sh.at[rs], psem.at[0]).wait()
    pltpu.make_async_copy(zeros_hbm.at[rs], accum_sh.at[rs], psem.at[1]).wait()
    pltpu.make_async_copy(ridx_hbm.at[t], ridx_v, psem.at[2]).wait()
    pltpu.make_async_copy(cidx_hbm.at[t], cidx_v, psem.at[3]).wait()
    plsc.subcore_barrier()

    def gsrc(b):
        return g_sh

    for b in range(NBUF):
        pltpu.async_copy(gsrc(b).at[ridx_v.at[b]], rows_v.at[b], sem.at[b])

    def chunk(j0, carry):
        for b in range(NBUF):
            j = j0 * NBUF + b
            pltpu.make_async_copy(gsrc(b).at[ridx_v.at[j]], rows_v.at[b],
                                  sem.at[b]).wait()
            pltpu.sync_copy(rows_v.at[b], accum_sh.at[cidx_v.at[j]], add=True)
            pltpu.async_copy(gsrc(b).at[ridx_v.at[j + NBUF]], rows_v.at[b],
                             sem.at[b])
        return carry

    lax.fori_loop(0, CPT // NBUF - 1, chunk, 0)
    for b in range(NBUF):
        j = CPT - NBUF + b
        pltpu.make_async_copy(gsrc(b).at[ridx_v.at[j]], rows_v.at[b],
                              sem.at[b]).wait()
        pltpu.sync_copy(rows_v.at[b], accum_sh.at[cidx_v.at[j]], add=True)
    plsc.subcore_barrier()
    pltpu.sync_copy(accum_sh.at[pl.ds(s * RPT, RPT)], out_hbm.at[c, pl.ds(s * RPT, RPT)])


@functools.cache
def _get_sc_hop():
    return pl.kernel(
        _sc_hop_body,
        out_type=jax.ShapeDtypeStruct((2, NPAD, OUT_CH), jnp.float32),
        mesh=plsc.VectorSubcoreMesh(core_axis_name="c", subcore_axis_name="s"),
        compiler_params=pltpu.CompilerParams(use_tc_tiling_on_sc=False),
        scratch_types=[
            pltpu.VMEM((CPT, CHUNK), jnp.int32),
            pltpu.VMEM((CPT, CHUNK), jnp.int32),
            pltpu.VMEM((NBUF, CHUNK, OUT_CH), jnp.float32),
            pltpu.VMEM_SHARED((NPAD, OUT_CH), jnp.float32),
            pltpu.VMEM_SHARED((NPAD, OUT_CH), jnp.float32),
            pltpu.SemaphoreType.DMA((NBUF,)),
            pltpu.SemaphoreType.DMA((4,)),
        ],
    )


def _mlp_body(x_ref, w1t_ref, b1_ref, w2t_ref, b2_ref, degp_ref,
              g0_ref, dis2_ref, disinv_ref):
    h1 = jnp.dot(x_ref[...], w1t_ref[...], preferred_element_type=jnp.float32)
    h1 = jnp.maximum(h1 + b1_ref[...], 0.0)
    h0 = jnp.dot(h1, w2t_ref[...], preferred_element_type=jnp.float32) + b2_ref[...]
    deg = 1.0 + degp_ref[0, :, 0:1] + degp_ref[1, :, 0:1]
    dis = lax.rsqrt(deg)
    g0_ref[...] = dis * h0
    dis2_ref[...] = jnp.broadcast_to(1.0 / deg, (NB, OUT_CH))
    disinv_ref[...] = jnp.broadcast_to(jnp.sqrt(deg), (NB, OUT_CH))


def _mlp_call(x_pad, w1t, b1, w2t, b2, degp):
    return pl.pallas_call(
        _mlp_body,
        grid=(GRID,),
        in_specs=[
            pl.BlockSpec((NB, IN_CH), lambda i: (i, 0)),
            pl.BlockSpec((IN_CH, HID_CH), lambda i: (0, 0)),
            pl.BlockSpec((1, HID_CH), lambda i: (0, 0)),
            pl.BlockSpec((HID_CH, OUT_CH), lambda i: (0, 0)),
            pl.BlockSpec((1, OUT_CH), lambda i: (0, 0)),
            pl.BlockSpec((2, NB, 8), lambda i: (0, i, 0)),
        ],
        out_specs=[
            pl.BlockSpec((NB, OUT_CH), lambda i: (i, 0)),
            pl.BlockSpec((NB, OUT_CH), lambda i: (i, 0)),
            pl.BlockSpec((NB, OUT_CH), lambda i: (i, 0)),
        ],
        out_shape=[
            jax.ShapeDtypeStruct((NPAD, OUT_CH), jnp.float32),
            jax.ShapeDtypeStruct((NPAD, OUT_CH), jnp.float32),
            jax.ShapeDtypeStruct((NPAD, OUT_CH), jnp.float32),
        ],
    )(x_pad, w1t, b1, w2t, b2, degp)


def _combine_body(acc_ref, g_ref, dis2_ref, g_out_ref):
    g_out_ref[...] = dis2_ref[...] * (acc_ref[0] + acc_ref[1] + g_ref[...])


def _combine_call(acc, g, dis2):
    return pl.pallas_call(
        _combine_body,
        grid=(GRID,),
        in_specs=[
            pl.BlockSpec((2, NB, OUT_CH), lambda i: (0, i, 0)),
            pl.BlockSpec((NB, OUT_CH), lambda i: (i, 0)),
            pl.BlockSpec((NB, OUT_CH), lambda i: (i, 0)),
        ],
        out_specs=pl.BlockSpec((NB, OUT_CH), lambda i: (i, 0)),
        out_shape=jax.ShapeDtypeStruct((NPAD, OUT_CH), jnp.float32),
    )(acc, g, dis2)


def _final_body(coes_ref, acc_ref, dis2_ref, disinv_ref, *refs):
    g_refs = refs[:P_HOPS]
    o_ref = refs[P_HOPS]
    g_last = g_refs[P_HOPS - 1][...]
    g_end = dis2_ref[...] * (acc_ref[0] + acc_ref[1] + g_last)
    acc_h = coes_ref[P_HOPS] * g_end
    for q in range(P_HOPS):
        acc_h = acc_h + coes_ref[q] * g_refs[q][...]
    h = disinv_ref[...] * acc_h
    m = jnp.max(h, axis=1, keepdims=True)
    e = jnp.exp(h - m)
    o_ref[...] = h - m - jnp.log(jnp.sum(e, axis=1, keepdims=True))


def _final_call(coes, acc, dis2, disinv, g_list):
    blk = pl.BlockSpec((NB, OUT_CH), lambda i: (i, 0))
    return pl.pallas_call(
        _final_body,
        grid=(GRID,),
        in_specs=[
            pl.BlockSpec(memory_space=pltpu.SMEM),
            pl.BlockSpec((2, NB, OUT_CH), lambda i: (0, i, 0)),
            blk, blk,
        ] + [blk] * P_HOPS,
        out_specs=blk,
        out_shape=jax.ShapeDtypeStruct((NPAD, OUT_CH), jnp.float32),
    )(coes, acc, dis2, disinv, *g_list)


def kernel(x, edge_index, W1, b1, W2, b2, coes):
    row = edge_index[0].astype(jnp.int32)
    col = edge_index[1].astype(jnp.int32)
    pad = E_PAD - N_EDGES
    row_p = jnp.concatenate([row, jnp.zeros((pad,), jnp.int32)]).reshape(
        NTILES, CPT, CHUNK)
    col_p = jnp.concatenate([col, jnp.full((pad,), TRASH, jnp.int32)]).reshape(
        NTILES, CPT, CHUNK)
    zeros8 = jnp.zeros((NPAD, 8), jnp.float32)
    ones8 = jnp.ones((CHUNK, 8), jnp.float32)
    zeros64 = jnp.zeros((NPAD, OUT_CH), jnp.float32)
    x_pad = jnp.zeros((NPAD, IN_CH), jnp.float32).at[:N_NODES].set(x)

    degp = _get_sc_deg()(col_p, ones8, zeros8)
    g, dis2, disinv = _mlp_call(x_pad, W1.T, b1.reshape(1, -1),
                                W2.T, b2.reshape(1, -1), degp)
    g_list = [g]
    for p in range(P_HOPS):
        acc = _get_sc_hop()(g, row_p, col_p, zeros64)
        if p < P_HOPS - 1:
            g = _combine_call(acc, g, dis2)
            g_list.append(g)
    out = _final_call(coes, acc, dis2, disinv, g_list)
    return out[:N_NODES]
